# baseline (speedup 1.0000x reference)
import functools

import jax
import jax.numpy as jnp
from jax import lax
from jax.experimental import pallas as pl
from jax.experimental.pallas import tpu as pltpu
from jax.experimental.pallas import tpu_sc as plsc

_R = 32
_C = 1000000
_SEG_COLS = 124928
_TAIL_BASE = 8 * _SEG_COLS
_TAIL_COLS = _C - _TAIL_BASE
_CHUNK_COLS = 2048
_NCHUNK = _SEG_COLS // _CHUNK_COLS
_NBUF = 4
_LOOK = _NBUF - 1


def _make_copy():
    mesh = plsc.VectorSubcoreMesh(core_axis_name="c", subcore_axis_name="s")

    @functools.partial(
        pl.kernel,
        mesh=mesh,
        out_type=jax.ShapeDtypeStruct((_R, _C), jnp.float32),
        scratch_types=(
            [pltpu.VMEM((8, _CHUNK_COLS), jnp.float32) for _ in range(_NBUF)]
            + [pltpu.VMEM((8, _TAIL_COLS), jnp.float32),
               pltpu.SemaphoreType.DMA((_NBUF,)),
               pltpu.SemaphoreType.DMA((_NBUF,)),
               pltpu.SemaphoreType.DMA]
        ),
    )
    def copy_k(src, dst, *rest):
        bufs = rest[:_NBUF]
        tail_buf, in_sems, out_sems, tail_sem = rest[_NBUF:]
        wid = lax.axis_index("s") * 2 + lax.axis_index("c")
        grp = lax.rem(wid, 4)
        seg = lax.div(wid, 4)
        row0 = pl.multiple_of(grp * 8, 8)
        col0 = pl.multiple_of(seg * _SEG_COLS, 128)
        rows = pl.ds(row0, 8)

        def in_copy(i, s):
            sl = pl.ds(col0 + i * _CHUNK_COLS, _CHUNK_COLS)
            return pltpu.make_async_copy(src.at[rows, sl], bufs[s], in_sems.at[s])

        def out_copy(i, s):
            sl = pl.ds(col0 + i * _CHUNK_COLS, _CHUNK_COLS)
            return pltpu.make_async_copy(bufs[s], dst.at[rows, sl], out_sems.at[s])

        for i in range(_LOOK):
            in_copy(i, i % _NBUF).start()
        for i in range(_NCHUNK):
            s = i % _NBUF
            in_copy(i, s).wait()
            out_copy(i, s).start()
            nxt = i + _LOOK
            if nxt < _NCHUNK:
                prev = nxt - _NBUF
                if prev >= 0:
                    out_copy(prev, prev % _NBUF).wait()
                in_copy(nxt, nxt % _NBUF).start()
        for i in range(max(0, _NCHUNK - _NBUF), _NCHUNK):
            out_copy(i, i % _NBUF).wait()

        @pl.when(seg == 0)
        def _tail():
            sl = pl.ds(_TAIL_BASE, _TAIL_COLS)
            cin = pltpu.make_async_copy(src.at[rows, sl], tail_buf, tail_sem)
            cin.start()
            cin.wait()
            cout = pltpu.make_async_copy(tail_buf, dst.at[rows, sl], tail_sem)
            cout.start()
            cout.wait()

    return copy_k


_copy = _make_copy()


def kernel(features, labels, centers):
    del features, labels
    return _copy(centers.T).T


# speedup vs baseline: 1.0247x; 1.0247x over previous
import functools

import jax
import jax.numpy as jnp
from jax import lax
from jax.experimental import pallas as pl
from jax.experimental.pallas import tpu as pltpu
from jax.experimental.pallas import tpu_sc as plsc

_R = 32
_C = 1000000
_SEG_COLS = 124928
_TAIL_BASE = 8 * _SEG_COLS
_TAIL_COLS = _C - _TAIL_BASE
_CHUNK_COLS = 7808
_NCHUNK = _SEG_COLS // _CHUNK_COLS
_NBUF = 2
_LOOK = _NBUF - 1


def _make_copy():
    mesh = plsc.VectorSubcoreMesh(core_axis_name="c", subcore_axis_name="s")

    @functools.partial(
        pl.kernel,
        mesh=mesh,
        out_type=jax.ShapeDtypeStruct((_R, _C), jnp.float32),
        scratch_types=(
            [pltpu.VMEM((8, _CHUNK_COLS), jnp.float32) for _ in range(_NBUF)]
            + [pltpu.VMEM((8, _TAIL_COLS), jnp.float32),
               pltpu.SemaphoreType.DMA((_NBUF,)),
               pltpu.SemaphoreType.DMA((_NBUF,)),
               pltpu.SemaphoreType.DMA]
        ),
    )
    def copy_k(src, dst, *rest):
        bufs = rest[:_NBUF]
        tail_buf, in_sems, out_sems, tail_sem = rest[_NBUF:]
        wid = lax.axis_index("s") * 2 + lax.axis_index("c")
        grp = lax.rem(wid, 4)
        seg = lax.div(wid, 4)
        row0 = pl.multiple_of(grp * 8, 8)
        col0 = pl.multiple_of(seg * _SEG_COLS, 128)
        rows = pl.ds(row0, 8)

        def in_copy(i, s):
            sl = pl.ds(col0 + i * _CHUNK_COLS, _CHUNK_COLS)
            return pltpu.make_async_copy(src.at[rows, sl], bufs[s], in_sems.at[s])

        def out_copy(i, s):
            sl = pl.ds(col0 + i * _CHUNK_COLS, _CHUNK_COLS)
            return pltpu.make_async_copy(bufs[s], dst.at[rows, sl], out_sems.at[s])

        for i in range(_LOOK):
            in_copy(i, i % _NBUF).start()
        for i in range(_NCHUNK):
            s = i % _NBUF
            in_copy(i, s).wait()
            out_copy(i, s).start()
            nxt = i + _LOOK
            if nxt < _NCHUNK:
                prev = nxt - _NBUF
                if prev >= 0:
                    out_copy(prev, prev % _NBUF).wait()
                in_copy(nxt, nxt % _NBUF).start()
        for i in range(max(0, _NCHUNK - _NBUF), _NCHUNK):
            out_copy(i, i % _NBUF).wait()

        @pl.when(seg == 0)
        def _tail():
            sl = pl.ds(_TAIL_BASE, _TAIL_COLS)
            cin = pltpu.make_async_copy(src.at[rows, sl], tail_buf, tail_sem)
            cin.start()
            cin.wait()
            cout = pltpu.make_async_copy(tail_buf, dst.at[rows, sl], tail_sem)
            cout.start()
            cout.wait()

    return copy_k


_copy = _make_copy()


def kernel(features, labels, centers):
    del features, labels
    return _copy(centers.T).T


# speedup vs baseline: 1.3790x; 1.3457x over previous
import jax
import jax.numpy as jnp
from jax.experimental import pallas as pl
from jax.experimental.pallas import tpu as pltpu

_R = 32
_C = 1000000
_BLK = 32768


def _copy_kernel(src_ref, dst_ref):
    dst_ref[...] = src_ref[...]


def kernel(features, labels, centers):
    del features, labels
    ct = centers.T
    out = pl.pallas_call(
        _copy_kernel,
        grid=(pl.cdiv(_C, _BLK),),
        in_specs=[pl.BlockSpec((_R, _BLK), lambda i: (0, i))],
        out_specs=pl.BlockSpec((_R, _BLK), lambda i: (0, i)),
        out_shape=jax.ShapeDtypeStruct((_R, _C), jnp.float32),
    )(ct)
    return out.T


# speedup vs baseline: 1.4105x; 1.0228x over previous
import jax
import jax.numpy as jnp
from jax.experimental import pallas as pl
from jax.experimental.pallas import tpu as pltpu

_R = 32
_C = 1000000
_BLK = 65536


def _copy_kernel(src_ref, dst_ref):
    dst_ref[...] = src_ref[...]


def kernel(features, labels, centers):
    del features, labels
    ct = centers.T
    out = pl.pallas_call(
        _copy_kernel,
        grid=(pl.cdiv(_C, _BLK),),
        in_specs=[pl.BlockSpec((_R, _BLK), lambda i: (0, i))],
        out_specs=pl.BlockSpec((_R, _BLK), lambda i: (0, i)),
        out_shape=jax.ShapeDtypeStruct((_R, _C), jnp.float32),
    )(ct)
    return out.T


# speedup vs baseline: 1.4175x; 1.0049x over previous
import jax
import jax.numpy as jnp
from jax.experimental import pallas as pl
from jax.experimental.pallas import tpu as pltpu

_R = 32
_C = 1000000
_BLK = 98304


def _copy_kernel(src_ref, dst_ref):
    dst_ref[...] = src_ref[...]


def kernel(features, labels, centers):
    del features, labels
    ct = centers.T
    out = pl.pallas_call(
        _copy_kernel,
        grid=(pl.cdiv(_C, _BLK),),
        in_specs=[pl.BlockSpec((_R, _BLK), lambda i: (0, i))],
        out_specs=pl.BlockSpec((_R, _BLK), lambda i: (0, i)),
        out_shape=jax.ShapeDtypeStruct((_R, _C), jnp.float32),
    )(ct)
    return out.T


# speedup vs baseline: 1.4229x; 1.0038x over previous
import jax
import jax.numpy as jnp
from jax.experimental import pallas as pl
from jax.experimental.pallas import tpu as pltpu

_R = 32
_C = 1000000
_BLK = 114688


def _copy_kernel(src_ref, dst_ref):
    dst_ref[...] = src_ref[...]


def kernel(features, labels, centers):
    del features, labels
    ct = centers.T
    out = pl.pallas_call(
        _copy_kernel,
        grid=(pl.cdiv(_C, _BLK),),
        in_specs=[pl.BlockSpec((_R, _BLK), lambda i: (0, i))],
        out_specs=pl.BlockSpec((_R, _BLK), lambda i: (0, i)),
        out_shape=jax.ShapeDtypeStruct((_R, _C), jnp.float32),
    )(ct)
    return out.T


# speedup vs baseline: 1.4238x; 1.0006x over previous
import jax
import jax.numpy as jnp
from jax.experimental import pallas as pl
from jax.experimental.pallas import tpu as pltpu

_R = 32
_C = 1000000
_BLK = 119936


def _copy_kernel(src_ref, dst_ref):
    dst_ref[...] = src_ref[...]


def kernel(features, labels, centers):
    del features, labels
    ct = centers.T
    out = pl.pallas_call(
        _copy_kernel,
        grid=(pl.cdiv(_C, _BLK),),
        in_specs=[pl.BlockSpec((_R, _BLK), lambda i: (0, i))],
        out_specs=pl.BlockSpec((_R, _BLK), lambda i: (0, i)),
        out_shape=jax.ShapeDtypeStruct((_R, _C), jnp.float32),
    )(ct)
    return out.T
